# baseline (speedup 1.0000x reference)
import functools

import jax
import jax.numpy as jnp
from jax import lax
from jax.experimental import pallas as pl
from jax.experimental.pallas import tpu as pltpu
from jax.experimental.pallas import tpu_sc as plsc

N = 10000
NPAD = 10240
E = 320000
DFEAT = 128
D1 = 16
D2 = 32

NC = 2
NS = 16
LANES = 16
CHUNK = 128
NCH_TILE = 80
NBUF = 8
E_PAD = NC * NS * NCH_TILE * CHUNK
TOT_CH = E_PAD // CHUNK
STRIPE = NPAD // NS
JUNK_ROW = N + 16

@functools.cache
def _mesh():
    return plsc.VectorSubcoreMesh(core_axis_name="c", subcore_axis_name="s",
                                  num_cores=NC, num_subcores=NS)


def _zero_fill(buf, nrows, width):
    @pl.loop(0, nrows)
    def _(i):
        for k in range(width // LANES):
            buf[i, pl.ds(k * LANES, LANES)] = jnp.zeros((LANES,), jnp.float32)


def _sc_degree(dst_hbm, out_hbm, acc, dst_v, ones_v, zbuf):
    c = lax.axis_index("c")
    s = lax.axis_index("s")
    wid = c * NS + s
    _zero_fill(zbuf, STRIPE, D1)

    @pl.loop(0, CHUNK)
    def _(i):
        ones_v[i, :] = jnp.ones((LANES,), jnp.float32)

    pltpu.sync_copy(zbuf, acc.at[pl.ds(s * STRIPE, STRIPE)])
    pltpu.sync_copy(dst_hbm.at[pl.ds(wid * NCH_TILE, NCH_TILE)], dst_v)
    plsc.subcore_barrier()

    @pl.loop(0, NCH_TILE)
    def _(j):
        pltpu.sync_copy(ones_v, acc.at[dst_v.at[j]], add=True)

    plsc.subcore_barrier()
    pltpu.sync_copy(acc.at[pl.ds(s * STRIPE, STRIPE)],
                    out_hbm.at[c, pl.ds(s * STRIPE, STRIPE)])


@functools.cache
def _degree_call():
    return pl.kernel(
        _sc_degree,
        out_type=jax.ShapeDtypeStruct((NC, NPAD, D1), jnp.float32),
        mesh=_mesh(),
        scratch_types=[
            pltpu.VMEM_SHARED((NPAD, D1), jnp.float32),
            pltpu.VMEM((NCH_TILE, CHUNK), jnp.int32),
            pltpu.VMEM((CHUNK, D1), jnp.float32),
            pltpu.VMEM((STRIPE, D1), jnp.float32),
        ],
    )


def _sc_agg(D, g_hbm, src_hbm, dst_hbm, out_hbm,
            acc, src_v, dst_v, rows, zbuf, *sems):
    gsem = sems[:NBUF]
    ssem = sems[NBUF:]
    c = lax.axis_index("c")
    s = lax.axis_index("s")
    wid = c * NS + s
    _zero_fill(zbuf, STRIPE, D)
    pltpu.sync_copy(zbuf, acc.at[pl.ds(s * STRIPE, STRIPE)])
    pltpu.sync_copy(src_hbm.at[pl.ds(wid * NCH_TILE, NCH_TILE)], src_v)
    pltpu.sync_copy(dst_hbm.at[pl.ds(wid * NCH_TILE, NCH_TILE)], dst_v)
    for b in range(NBUF):
        pltpu.async_copy(g_hbm.at[src_v.at[b]], rows.at[b], gsem[b])
    plsc.subcore_barrier()

    @pl.loop(0, NCH_TILE // NBUF)
    def _(gi):
        for b in range(NBUF):
            j = gi * NBUF + b
            pltpu.make_async_copy(g_hbm.at[src_v.at[j]], rows.at[b],
                                  gsem[b]).wait()
            pltpu.async_copy(rows.at[b], acc.at[dst_v.at[j]], ssem[b],
                             add=True)
            bp = (b - 1) % NBUF
            jp = j - 1

            @pl.when(jnp.logical_and(jp >= 0, jp + NBUF < NCH_TILE))
            def _():
                pltpu.make_async_copy(rows.at[bp], acc.at[dst_v.at[0]],
                                      ssem[bp]).wait()
                pltpu.async_copy(g_hbm.at[src_v.at[jp + NBUF]], rows.at[bp],
                                 gsem[bp])

    for b in range(NBUF):
        pltpu.make_async_copy(rows.at[b], acc.at[dst_v.at[0]],
                              ssem[b]).wait()
    plsc.subcore_barrier()
    pltpu.sync_copy(acc.at[pl.ds(s * STRIPE, STRIPE)],
                    out_hbm.at[c, pl.ds(s * STRIPE, STRIPE)])


@functools.cache
def _make_agg_call(D):
    return pl.kernel(
        functools.partial(_sc_agg, D),
        out_type=jax.ShapeDtypeStruct((NC, NPAD, D), jnp.float32),
        mesh=_mesh(),
        compiler_params=pltpu.CompilerParams(use_tc_tiling_on_sc=False),
        scratch_types=[
            pltpu.VMEM_SHARED((NPAD, D), jnp.float32),
            pltpu.VMEM((NCH_TILE, CHUNK), jnp.int32),
            pltpu.VMEM((NCH_TILE, CHUNK), jnp.int32),
            pltpu.VMEM((NBUF, CHUNK, D), jnp.float32),
            pltpu.VMEM((STRIPE, D), jnp.float32),
        ] + [pltpu.SemaphoreType.DMA] * (2 * NBUF),
    )


def _tc_pre(deg_ref, x_ref, w1_ref, g1_ref, dis_ref):
    deg = deg_ref[0, :N, 0:1] + deg_ref[1, :N, 0:1] + 1.0
    dis = lax.rsqrt(deg)
    h = jnp.dot(x_ref[...], w1_ref[...], preferred_element_type=jnp.float32)
    g1_ref[...] = h * dis
    dis_ref[...] = dis


def _tc_mid(s1_ref, g1_ref, dis_ref, b1_ref, w2_ref, g2_ref):
    dis = dis_ref[...]
    a1 = (s1_ref[0, :N, :] + s1_ref[1, :N, :] + g1_ref[...]) * dis \
        + b1_ref[...][None, :]
    z = jnp.maximum(a1, 0.0)
    h2 = jnp.dot(z, w2_ref[...], preferred_element_type=jnp.float32)
    g2_ref[...] = h2 * dis


def _tc_post(s2_ref, g2_ref, dis_ref, b2_ref, o_ref):
    a2 = (s2_ref[0, :N, :] + s2_ref[1, :N, :] + g2_ref[...]) * dis_ref[...] \
        + b2_ref[...][None, :]
    m = jnp.max(a2, axis=1, keepdims=True)
    lse = jnp.log(jnp.sum(jnp.exp(a2 - m), axis=1, keepdims=True)) + m
    o_ref[...] = a2 - lse


_tc_pre_call = pl.pallas_call(
    _tc_pre,
    out_shape=[jax.ShapeDtypeStruct((N, D1), jnp.float32),
               jax.ShapeDtypeStruct((N, 1), jnp.float32)],
)

_tc_mid_call = pl.pallas_call(
    _tc_mid,
    out_shape=jax.ShapeDtypeStruct((N, D2), jnp.float32),
)

_tc_post_call = pl.pallas_call(
    _tc_post,
    out_shape=jax.ShapeDtypeStruct((N, D2), jnp.float32),
)


def kernel(x, edge_index, W1, b1, W2, b2):
    src = edge_index[0].astype(jnp.int32)
    dst = edge_index[1].astype(jnp.int32)
    n_extra = E_PAD - E
    src_p = jnp.concatenate(
        [src, jnp.zeros((n_extra,), jnp.int32)]).reshape(TOT_CH, CHUNK)
    dst_p = jnp.concatenate(
        [dst, jnp.full((n_extra,), JUNK_ROW, jnp.int32)]).reshape(TOT_CH, CHUNK)

    deg_parts = _degree_call()(dst_p)
    g1, dis = _tc_pre_call(deg_parts, x, W1)
    s1 = _make_agg_call(D1)(g1, src_p, dst_p)
    g2 = _tc_mid_call(s1, g1, dis, b1, W2)
    s2 = _make_agg_call(D2)(g2, src_p, dst_p)
    return _tc_post_call(s2, g2, dis, b2)


# speedup vs baseline: 1.6160x; 1.6160x over previous
import functools

import jax
import jax.numpy as jnp
from jax import lax
from jax.experimental import pallas as pl
from jax.experimental.pallas import tpu as pltpu
from jax.experimental.pallas import tpu_sc as plsc

N = 10000
NPAD = 10240
E = 320000
DFEAT = 128
D1 = 16
D2 = 32

NC = 2
NS = 16
LANES = 16
CHUNK = 128
NCH_TILE = 80
NBUF = 8
E_PAD = NC * NS * NCH_TILE * CHUNK
TOT_CH = E_PAD // CHUNK
STRIPE = NPAD // NS
JUNK_ROW = N + 16

@functools.cache
def _mesh():
    return plsc.VectorSubcoreMesh(core_axis_name="c", subcore_axis_name="s",
                                  num_cores=NC, num_subcores=NS)


def _zero_fill(buf, nrows, width):
    @pl.loop(0, nrows)
    def _(i):
        for k in range(width // LANES):
            buf[i, pl.ds(k * LANES, LANES)] = jnp.zeros((LANES,), jnp.float32)


def _sc_degree(dst_hbm, out_hbm, acc, dst_v, ones_v, zbuf):
    c = lax.axis_index("c")
    s = lax.axis_index("s")
    wid = c * NS + s
    _zero_fill(zbuf, STRIPE, D1)

    @pl.loop(0, CHUNK)
    def _(i):
        ones_v[i, :] = jnp.ones((LANES,), jnp.float32)

    pltpu.sync_copy(zbuf, acc.at[pl.ds(s * STRIPE, STRIPE)])
    pltpu.sync_copy(dst_hbm.at[pl.ds(wid * NCH_TILE, NCH_TILE)], dst_v)
    plsc.subcore_barrier()

    @pl.loop(0, NCH_TILE)
    def _(j):
        pltpu.sync_copy(ones_v, acc.at[dst_v.at[j]], add=True)

    plsc.subcore_barrier()
    pltpu.sync_copy(acc.at[pl.ds(s * STRIPE, STRIPE)],
                    out_hbm.at[c, pl.ds(s * STRIPE, STRIPE)])


@functools.cache
def _degree_call():
    return pl.kernel(
        _sc_degree,
        out_type=jax.ShapeDtypeStruct((NC, NPAD, D1), jnp.float32),
        mesh=_mesh(),
        scratch_types=[
            pltpu.VMEM_SHARED((NPAD, D1), jnp.float32),
            pltpu.VMEM((NCH_TILE, CHUNK), jnp.int32),
            pltpu.VMEM((CHUNK, D1), jnp.float32),
            pltpu.VMEM((STRIPE, D1), jnp.float32),
        ],
    )


def _sc_agg(D, g_hbm, src_hbm, dst_hbm, out_hbm,
            acc, g_sp, src_v, dst_v, rows, zbuf, *sems):
    gsem = sems[:NBUF]
    ssem = sems[NBUF:]
    c = lax.axis_index("c")
    s = lax.axis_index("s")
    wid = c * NS + s
    _zero_fill(zbuf, STRIPE, D)
    pltpu.sync_copy(zbuf, acc.at[pl.ds(s * STRIPE, STRIPE)])
    pltpu.sync_copy(g_hbm.at[pl.ds(s * (N // NS), N // NS)],
                    g_sp.at[pl.ds(s * (N // NS), N // NS)])
    pltpu.sync_copy(src_hbm.at[pl.ds(wid * NCH_TILE, NCH_TILE)], src_v)
    pltpu.sync_copy(dst_hbm.at[pl.ds(wid * NCH_TILE, NCH_TILE)], dst_v)
    plsc.subcore_barrier()
    for b in range(NBUF):
        pltpu.async_copy(g_sp.at[src_v.at[b]], rows.at[b], gsem[b])

    @pl.loop(0, NCH_TILE // NBUF)
    def _(gi):
        for b in range(NBUF):
            j = gi * NBUF + b
            pltpu.make_async_copy(g_sp.at[src_v.at[j]], rows.at[b],
                                  gsem[b]).wait()
            pltpu.async_copy(rows.at[b], acc.at[dst_v.at[j]], ssem[b],
                             add=True)
            bp = (b - 1) % NBUF
            jp = j - 1

            @pl.when(jnp.logical_and(jp >= 0, jp + NBUF < NCH_TILE))
            def _():
                pltpu.make_async_copy(rows.at[bp], acc.at[dst_v.at[0]],
                                      ssem[bp]).wait()
                pltpu.async_copy(g_sp.at[src_v.at[jp + NBUF]], rows.at[bp],
                                 gsem[bp])

    for b in range(NBUF):
        pltpu.make_async_copy(rows.at[b], acc.at[dst_v.at[0]],
                              ssem[b]).wait()
    plsc.subcore_barrier()
    pltpu.sync_copy(acc.at[pl.ds(s * STRIPE, STRIPE)],
                    out_hbm.at[c, pl.ds(s * STRIPE, STRIPE)])


@functools.cache
def _make_agg_call(D):
    return pl.kernel(
        functools.partial(_sc_agg, D),
        out_type=jax.ShapeDtypeStruct((NC, NPAD, D), jnp.float32),
        mesh=_mesh(),
        compiler_params=pltpu.CompilerParams(use_tc_tiling_on_sc=False),
        scratch_types=[
            pltpu.VMEM_SHARED((NPAD, D), jnp.float32),
            pltpu.VMEM_SHARED((N, D), jnp.float32),
            pltpu.VMEM((NCH_TILE, CHUNK), jnp.int32),
            pltpu.VMEM((NCH_TILE, CHUNK), jnp.int32),
            pltpu.VMEM((NBUF, CHUNK, D), jnp.float32),
            pltpu.VMEM((STRIPE, D), jnp.float32),
        ] + [pltpu.SemaphoreType.DMA] * (2 * NBUF),
    )


def _tc_pre(deg_ref, x_ref, w1_ref, g1_ref, dis_ref):
    deg = deg_ref[0, :N, 0:1] + deg_ref[1, :N, 0:1] + 1.0
    dis = lax.rsqrt(deg)
    h = jnp.dot(x_ref[...], w1_ref[...], preferred_element_type=jnp.float32)
    g1_ref[...] = h * dis
    dis_ref[...] = dis


def _tc_mid(s1_ref, g1_ref, dis_ref, b1_ref, w2_ref, g2_ref):
    dis = dis_ref[...]
    a1 = (s1_ref[0, :N, :] + s1_ref[1, :N, :] + g1_ref[...]) * dis \
        + b1_ref[...][None, :]
    z = jnp.maximum(a1, 0.0)
    h2 = jnp.dot(z, w2_ref[...], preferred_element_type=jnp.float32)
    g2_ref[...] = h2 * dis


def _tc_post(s2_ref, g2_ref, dis_ref, b2_ref, o_ref):
    a2 = (s2_ref[0, :N, :] + s2_ref[1, :N, :] + g2_ref[...]) * dis_ref[...] \
        + b2_ref[...][None, :]
    m = jnp.max(a2, axis=1, keepdims=True)
    lse = jnp.log(jnp.sum(jnp.exp(a2 - m), axis=1, keepdims=True)) + m
    o_ref[...] = a2 - lse


_tc_pre_call = pl.pallas_call(
    _tc_pre,
    out_shape=[jax.ShapeDtypeStruct((N, D1), jnp.float32),
               jax.ShapeDtypeStruct((N, 1), jnp.float32)],
)

_tc_mid_call = pl.pallas_call(
    _tc_mid,
    out_shape=jax.ShapeDtypeStruct((N, D2), jnp.float32),
)

_tc_post_call = pl.pallas_call(
    _tc_post,
    out_shape=jax.ShapeDtypeStruct((N, D2), jnp.float32),
)


def kernel(x, edge_index, W1, b1, W2, b2):
    src = edge_index[0].astype(jnp.int32)
    dst = edge_index[1].astype(jnp.int32)
    n_extra = E_PAD - E
    src_p = jnp.concatenate(
        [src, jnp.zeros((n_extra,), jnp.int32)]).reshape(TOT_CH, CHUNK)
    dst_p = jnp.concatenate(
        [dst, jnp.full((n_extra,), JUNK_ROW, jnp.int32)]).reshape(TOT_CH, CHUNK)

    deg_parts = _degree_call()(dst_p)
    g1, dis = _tc_pre_call(deg_parts, x, W1)
    s1 = _make_agg_call(D1)(g1, src_p, dst_p)
    g2 = _tc_mid_call(s1, g1, dis, b1, W2)
    s2 = _make_agg_call(D2)(g2, src_p, dst_p)
    return _tc_post_call(s2, g2, dis, b2)


# speedup vs baseline: 1.7131x; 1.0601x over previous
import functools

import jax
import jax.numpy as jnp
from jax import lax
from jax.experimental import pallas as pl
from jax.experimental.pallas import tpu as pltpu
from jax.experimental.pallas import tpu_sc as plsc

N = 10000
NPAD = 10240
E = 320000
DFEAT = 128
D1 = 16
D2 = 32

NC = 2
NS = 16
LANES = 16
CHUNK = 125
NCH_TILE = 80
NBUF = 8
TOT_CH = E // CHUNK
STRIPE = NPAD // NS

@functools.cache
def _mesh():
    return plsc.VectorSubcoreMesh(core_axis_name="c", subcore_axis_name="s",
                                  num_cores=NC, num_subcores=NS)


def _zero_fill(buf, nrows, width):
    @pl.loop(0, nrows)
    def _(i):
        for k in range(width // LANES):
            buf[i, pl.ds(k * LANES, LANES)] = jnp.zeros((LANES,), jnp.float32)


def _sc_degree(dst_hbm, out_hbm, acc, dst_v, ones_v, zbuf):
    c = lax.axis_index("c")
    s = lax.axis_index("s")
    wid = c * NS + s
    _zero_fill(zbuf, STRIPE, D1)

    @pl.loop(0, CHUNK)
    def _(i):
        ones_v[i, :] = jnp.ones((LANES,), jnp.float32)

    pltpu.sync_copy(zbuf, acc.at[pl.ds(s * STRIPE, STRIPE)])
    pltpu.sync_copy(dst_hbm.at[pl.ds(wid * NCH_TILE, NCH_TILE)], dst_v)
    plsc.subcore_barrier()

    @pl.loop(0, NCH_TILE)
    def _(j):
        pltpu.sync_copy(ones_v, acc.at[dst_v.at[j]], add=True)

    plsc.subcore_barrier()
    pltpu.sync_copy(acc.at[pl.ds(s * STRIPE, STRIPE)],
                    out_hbm.at[c, pl.ds(s * STRIPE, STRIPE)])


@functools.cache
def _degree_call():
    return pl.kernel(
        _sc_degree,
        out_type=jax.ShapeDtypeStruct((NC, NPAD, D1), jnp.float32),
        mesh=_mesh(),
        scratch_types=[
            pltpu.VMEM_SHARED((NPAD, D1), jnp.float32),
            pltpu.VMEM((NCH_TILE, CHUNK), jnp.int32),
            pltpu.VMEM((CHUNK, D1), jnp.float32),
            pltpu.VMEM((STRIPE, D1), jnp.float32),
        ],
    )


def _sc_agg(D, g_hbm, src_hbm, dst_hbm, out_hbm,
            acc, g_sp, src_v, dst_v, rows, zbuf, *sems):
    gsem = sems[:NBUF]
    ssem = sems[NBUF:]
    c = lax.axis_index("c")
    s = lax.axis_index("s")
    wid = c * NS + s
    _zero_fill(zbuf, STRIPE, D)
    pltpu.sync_copy(zbuf, acc.at[pl.ds(s * STRIPE, STRIPE)])
    pltpu.sync_copy(g_hbm.at[pl.ds(s * (N // NS), N // NS)],
                    g_sp.at[pl.ds(s * (N // NS), N // NS)])
    pltpu.sync_copy(src_hbm.at[pl.ds(wid * NCH_TILE, NCH_TILE)], src_v)
    pltpu.sync_copy(dst_hbm.at[pl.ds(wid * NCH_TILE, NCH_TILE)], dst_v)
    plsc.subcore_barrier()
    for b in range(NBUF):
        pltpu.async_copy(g_sp.at[src_v.at[b]], rows.at[b], gsem[b])

    @pl.loop(0, NCH_TILE // NBUF)
    def _(gi):
        for b in range(NBUF):
            j = gi * NBUF + b
            pltpu.make_async_copy(g_sp.at[src_v.at[j]], rows.at[b],
                                  gsem[b]).wait()
            pltpu.async_copy(rows.at[b], acc.at[dst_v.at[j]], ssem[b],
                             add=True)
            bp = (b - 1) % NBUF
            jp = j - 1

            @pl.when(jnp.logical_and(jp >= 0, jp + NBUF < NCH_TILE))
            def _():
                pltpu.make_async_copy(rows.at[bp], acc.at[dst_v.at[0]],
                                      ssem[bp]).wait()
                pltpu.async_copy(g_sp.at[src_v.at[jp + NBUF]], rows.at[bp],
                                 gsem[bp])

    for b in range(NBUF):
        pltpu.make_async_copy(rows.at[b], acc.at[dst_v.at[0]],
                              ssem[b]).wait()
    plsc.subcore_barrier()
    pltpu.sync_copy(acc.at[pl.ds(s * STRIPE, STRIPE)],
                    out_hbm.at[c, pl.ds(s * STRIPE, STRIPE)])


@functools.cache
def _make_agg_call(D):
    return pl.kernel(
        functools.partial(_sc_agg, D),
        out_type=jax.ShapeDtypeStruct((NC, NPAD, D), jnp.float32),
        mesh=_mesh(),
        compiler_params=pltpu.CompilerParams(use_tc_tiling_on_sc=False),
        scratch_types=[
            pltpu.VMEM_SHARED((NPAD, D), jnp.float32),
            pltpu.VMEM_SHARED((N, D), jnp.float32),
            pltpu.VMEM((NCH_TILE, CHUNK), jnp.int32),
            pltpu.VMEM((NCH_TILE, CHUNK), jnp.int32),
            pltpu.VMEM((NBUF, CHUNK, D), jnp.float32),
            pltpu.VMEM((STRIPE, D), jnp.float32),
        ] + [pltpu.SemaphoreType.DMA] * (2 * NBUF),
    )


def _tc_h1(x_ref, w1_ref, h_ref):
    h_ref[...] = jnp.dot(x_ref[...], w1_ref[...],
                         preferred_element_type=jnp.float32)


def _tc_g1(deg_ref, h_ref, g1_ref, dis_ref):
    deg = deg_ref[0, :N, 0:1] + deg_ref[1, :N, 0:1] + 1.0
    dis = lax.rsqrt(deg)
    g1_ref[...] = h_ref[...] * dis
    dis_ref[...] = dis


def _tc_mid(s1_ref, g1_ref, dis_ref, b1_ref, w2_ref, g2_ref):
    dis = dis_ref[...]
    a1 = (s1_ref[0, :N, :] + s1_ref[1, :N, :] + g1_ref[...]) * dis \
        + b1_ref[...][None, :]
    z = jnp.maximum(a1, 0.0)
    h2 = jnp.dot(z, w2_ref[...], preferred_element_type=jnp.float32)
    g2_ref[...] = h2 * dis


def _tc_post(s2_ref, g2_ref, dis_ref, b2_ref, o_ref):
    a2 = (s2_ref[0, :N, :] + s2_ref[1, :N, :] + g2_ref[...]) * dis_ref[...] \
        + b2_ref[...][None, :]
    m = jnp.max(a2, axis=1, keepdims=True)
    lse = jnp.log(jnp.sum(jnp.exp(a2 - m), axis=1, keepdims=True)) + m
    o_ref[...] = a2 - lse


_tc_h1_call = pl.pallas_call(
    _tc_h1,
    out_shape=jax.ShapeDtypeStruct((N, D1), jnp.float32),
)

_tc_g1_call = pl.pallas_call(
    _tc_g1,
    out_shape=[jax.ShapeDtypeStruct((N, D1), jnp.float32),
               jax.ShapeDtypeStruct((N, 1), jnp.float32)],
)

_tc_mid_call = pl.pallas_call(
    _tc_mid,
    out_shape=jax.ShapeDtypeStruct((N, D2), jnp.float32),
)

_tc_post_call = pl.pallas_call(
    _tc_post,
    out_shape=jax.ShapeDtypeStruct((N, D2), jnp.float32),
)


def kernel(x, edge_index, W1, b1, W2, b2):
    src_p = edge_index[0].astype(jnp.int32).reshape(TOT_CH, CHUNK)
    dst_p = edge_index[1].astype(jnp.int32).reshape(TOT_CH, CHUNK)

    h = _tc_h1_call(x, W1)
    deg_parts = _degree_call()(dst_p)
    g1, dis = _tc_g1_call(deg_parts, h)
    s1 = _make_agg_call(D1)(g1, src_p, dst_p)
    g2 = _tc_mid_call(s1, g1, dis, b1, W2)
    s2 = _make_agg_call(D2)(g2, src_p, dst_p)
    return _tc_post_call(s2, g2, dis, b2)
